# baseline (speedup 1.0000x reference)
import functools

import jax
import jax.numpy as jnp
from jax import lax
from jax.experimental import pallas as pl
from jax.experimental.pallas import tpu as pltpu
from jax.experimental.pallas import tpu_sc as plsc

N = 8192
D = 2048
E = 16
CAPMAX = 1024
R = E * CAPMAX
L = 16
NTILES = 32
TOK_A = N // L
GROUPS_A = TOK_A // L
ROWS_B = R // NTILES
K = 16

_mesh = plsc.VectorSubcoreMesh(core_axis_name="c", subcore_axis_name="s",
                               num_cores=2, num_subcores=16)
_params = pltpu.CompilerParams(needs_layout_passes=False)


def _fused_body(logits_hbm, x_hbm, zsrc_hbm, out_hbm,
                logits_v, dst_v, pos_v, cnt_v, off_v, counts_v,
                dst_all, pos_all, perm_v, buf0, buf1, zbuf, zidx_v,
                counts_sh, dst_sh, pos_sh,
                semg0, semg1, sems0, sems1, zsem):
    cid = lax.axis_index("c")
    sid = lax.axis_index("s")
    iota = lax.iota(jnp.int32, L)
    zvec = jnp.zeros((L,), jnp.int32)

    tbase = sid * TOK_A
    pltpu.sync_copy(logits_hbm.at[pl.ds(tbase * E, TOK_A * E)], logits_v)
    pltpu.sync_copy(zsrc_hbm, zbuf)
    cnt_v[...] = zvec

    def group(g, _):
        row0 = g * L
        flat = (iota + row0) * E
        cols = [plsc.load_gather(logits_v, [flat + d]) for d in range(E)]
        m = cols[0]
        for d in range(1, E):
            m = jnp.maximum(m, cols[d])
        dst = jnp.full((L,), E, jnp.int32)
        for d in range(E - 1, -1, -1):
            dst = jnp.where(cols[d] == m, d, dst)
        basec = plsc.load_gather(cnt_v, [dst])
        rank = zvec
        inc = zvec
        for d in range(E):
            md = dst == d
            mi = md.astype(jnp.int32)
            c = plsc.cumsum(mi)
            rank = jnp.where(md, rank + c - 1, rank)
            inc = jnp.where(iota == d, jnp.sum(mi), inc)
        cnt_v[...] = cnt_v[...] + inc
        dst_v[pl.ds(row0, L)] = dst
        pos_v[pl.ds(row0, L)] = basec + rank
        return 0

    lax.fori_loop(0, GROUPS_A, group, 0)

    pltpu.sync_copy(cnt_v, counts_sh.at[pl.ds(sid * L, L)])
    plsc.subcore_barrier()
    pltpu.sync_copy(counts_sh, counts_v)

    off = zvec
    tot = zvec
    sidv = jnp.full((L,), sid, jnp.int32)
    for t in range(L):
        row = counts_v[pl.ds(t * L, L)]
        off = off + jnp.where(sidv > t, row, zvec)
        tot = tot + row
    off_v[...] = off

    def add_off(g, _):
        row0 = g * L
        dstg = dst_v[pl.ds(row0, L)]
        pos_v[pl.ds(row0, L)] = (pos_v[pl.ds(row0, L)]
                                 + plsc.load_gather(off_v, [dstg]))
        return 0

    lax.fori_loop(0, GROUPS_A, add_off, 0)
    pltpu.sync_copy(dst_v, dst_sh.at[pl.ds(tbase, TOK_A)])
    pltpu.sync_copy(pos_v, pos_sh.at[pl.ds(tbase, TOK_A)])

    maxc = jnp.max(tot)
    cap = jnp.where(maxc <= 128, 128,
          jnp.where(maxc <= 256, 256,
          jnp.where(maxc <= 512, 512, CAPMAX))).astype(jnp.int32)
    cvec = jnp.minimum(tot, cap)

    plsc.subcore_barrier()

    wid = cid * 16 + sid
    base = wid * ROWS_B
    e = wid // (CAPMAX // ROWS_B)
    h0 = (wid % (CAPMAX // ROWS_B)) * ROWS_B
    ev = jnp.full((L,), e, jnp.int32)
    h0v = jnp.full((L,), h0, jnp.int32)
    v = jnp.sum(jnp.where(iota == ev, jnp.clip(cvec - h0v, 0, ROWS_B), zvec))
    nch = (v + K - 1) // K

    nzb = (ROWS_B - nch * K) // K

    def zdesc(j):
        start = pl.multiple_of(base + (nch + j) * K, 8)
        return pltpu.make_async_copy(zbuf, out_hbm.at[pl.ds(start, K)], zsem)

    def zfire(j, _):
        zdesc(j).start()
        return 0

    lax.fori_loop(0, nzb, zfire, 0)

    pltpu.sync_copy(dst_sh, dst_all)
    pltpu.sync_copy(pos_sh, pos_all)

    def initp(i, _):
        perm_v[pl.ds(i * L, L)] = zvec
        return 0

    lax.fori_loop(0, ROWS_B // L, initp, 0)

    def scan(g, _):
        t0 = g * L
        dstg = dst_all[pl.ds(t0, L)]
        posg = pos_all[pl.ds(t0, L)]
        rel = posg - h0v
        mask = (dstg == ev) & (rel >= 0) & (rel < ROWS_B)
        plsc.store_scatter(perm_v, [rel], iota + t0, mask=mask)
        return 0

    lax.fori_loop(0, N // L, scan, 0)

    KH = K // 2

    def gdescs(c, buf, semg):
        return [
            pltpu.make_async_copy(
                x_hbm.at[perm_v.at[pl.ds(c * K + h * KH, KH)]],
                buf.at[pl.ds(h * KH, KH)], semg)
            for h in range(2)
        ]

    def gstart(c, buf, semg):
        for d in gdescs(c, buf, semg):
            d.start()

    def gwait(c, buf, semg):
        for d in gdescs(c, buf, semg):
            d.wait()

    def sdesc(c, buf, sems):
        start = pl.multiple_of(base + c * K, 8)
        return pltpu.make_async_copy(buf, out_hbm.at[pl.ds(start, K)], sems)

    @pl.when(nch > 0)
    def _():
        gstart(0, buf0, semg0)

    def gstep(c, buf, semg, sems, obuf, osemg, osems):
        gwait(c, buf, semg)
        sdesc(c, buf, sems).start()

        @pl.when(c >= 1)
        def _():
            sdesc(c - 1, obuf, osems).wait()

        @pl.when(c + 1 < nch)
        def _():
            gstart(c + 1, obuf, osemg)

    def gbody(c, _):
        @pl.when(c % 2 == 0)
        def _():
            gstep(c, buf0, semg0, sems0, buf1, semg1, sems1)

        @pl.when(c % 2 == 1)
        def _():
            gstep(c, buf1, semg1, sems1, buf0, semg0, sems0)

        return 0

    lax.fori_loop(0, nch, gbody, 0)

    @pl.when((nch >= 1) & (nch % 2 == 1))
    def _():
        sdesc(nch - 1, buf0, sems0).wait()

    @pl.when((nch >= 1) & (nch % 2 == 0))
    def _():
        sdesc(nch - 1, buf1, sems1).wait()

    @pl.when(v < nch * K)
    def _():
        zidx_v[...] = jnp.minimum(base + v + iota, base + nch * K - 1)
        pltpu.async_copy(zbuf, out_hbm.at[zidx_v], zsem).wait()

    def zdrain(j, _):
        zdesc(j).wait()
        return 0

    lax.fori_loop(0, nzb, zdrain, 0)


_fused = functools.partial(
    pl.kernel,
    out_type=jax.ShapeDtypeStruct((R, D), jnp.float32),
    mesh=_mesh,
    scratch_types=[
        pltpu.VMEM((TOK_A * E,), jnp.float32),
        pltpu.VMEM((TOK_A,), jnp.int32),
        pltpu.VMEM((TOK_A,), jnp.int32),
        pltpu.VMEM((L,), jnp.int32),
        pltpu.VMEM((L,), jnp.int32),
        pltpu.VMEM((L * L,), jnp.int32),
        pltpu.VMEM((N,), jnp.int32),
        pltpu.VMEM((N,), jnp.int32),
        pltpu.VMEM((ROWS_B,), jnp.int32),
        pltpu.VMEM((K, D), jnp.float32),
        pltpu.VMEM((K, D), jnp.float32),
        pltpu.VMEM((L, D), jnp.float32),
        pltpu.VMEM((L,), jnp.int32),
        pltpu.VMEM_SHARED((L * L,), jnp.int32),
        pltpu.VMEM_SHARED((N,), jnp.int32),
        pltpu.VMEM_SHARED((N,), jnp.int32),
        pltpu.SemaphoreType.DMA,
        pltpu.SemaphoreType.DMA,
        pltpu.SemaphoreType.DMA,
        pltpu.SemaphoreType.DMA,
        pltpu.SemaphoreType.DMA,
    ],
    compiler_params=_params,
)(_fused_body)


def kernel(x, route_logits):
    zsrc = jnp.zeros((L, D), jnp.float32)
    out = _fused(route_logits.reshape(N * E), x, zsrc)
    return out.reshape(E, CAPMAX, D)


# speedup vs baseline: 1.1155x; 1.1155x over previous
import functools

import jax
import jax.numpy as jnp
from jax import lax
from jax.experimental import pallas as pl
from jax.experimental.pallas import tpu as pltpu
from jax.experimental.pallas import tpu_sc as plsc

N = 8192
D = 2048
E = 16
CAPMAX = 1024
R = E * CAPMAX
L = 16
NTILES = 32
TOK_A = N // L
GROUPS_A = TOK_A // L
ROWS_B = R // NTILES
K = 16

_mesh = plsc.VectorSubcoreMesh(core_axis_name="c", subcore_axis_name="s",
                               num_cores=2, num_subcores=16)
_params = pltpu.CompilerParams(needs_layout_passes=False)


def _fused_body(logits_hbm, x_hbm, zsrc_hbm, out_hbm,
                logits_v, dst_v, pos_v, cnt_v, off_v, counts_v,
                dst_all, pos_all, perm_v, buf0, buf1, zbuf, zidx_v,
                counts_sh, dst_sh, pos_sh,
                semg0, semg1, sems0, sems1, zsem):
    cid = lax.axis_index("c")
    sid = lax.axis_index("s")
    iota = lax.iota(jnp.int32, L)
    zvec = jnp.zeros((L,), jnp.int32)

    tbase = sid * TOK_A
    pltpu.sync_copy(logits_hbm.at[pl.ds(tbase * E, TOK_A * E)], logits_v)
    pltpu.sync_copy(zsrc_hbm, zbuf)
    cnt_v[...] = zvec

    def group(g, _):
        row0 = g * L
        flat = (iota + row0) * E
        cols = [plsc.load_gather(logits_v, [flat + d]) for d in range(E)]
        m = cols[0]
        for d in range(1, E):
            m = jnp.maximum(m, cols[d])
        dst = jnp.full((L,), E, jnp.int32)
        for d in range(E - 1, -1, -1):
            dst = jnp.where(cols[d] == m, d, dst)
        basec = plsc.load_gather(cnt_v, [dst])
        rank = zvec
        inc = zvec
        for d in range(E):
            md = dst == d
            mi = md.astype(jnp.int32)
            c = plsc.cumsum(mi)
            rank = jnp.where(md, rank + c - 1, rank)
            inc = jnp.where(iota == d, jnp.sum(mi), inc)
        cnt_v[...] = cnt_v[...] + inc
        dst_v[pl.ds(row0, L)] = dst
        pos_v[pl.ds(row0, L)] = basec + rank
        return 0

    lax.fori_loop(0, GROUPS_A, group, 0)

    pltpu.sync_copy(cnt_v, counts_sh.at[pl.ds(sid * L, L)])
    plsc.subcore_barrier()
    pltpu.sync_copy(counts_sh, counts_v)

    off = zvec
    tot = zvec
    sidv = jnp.full((L,), sid, jnp.int32)
    for t in range(L):
        row = counts_v[pl.ds(t * L, L)]
        off = off + jnp.where(sidv > t, row, zvec)
        tot = tot + row
    off_v[...] = off

    def add_off(g, _):
        row0 = g * L
        dstg = dst_v[pl.ds(row0, L)]
        pos_v[pl.ds(row0, L)] = (pos_v[pl.ds(row0, L)]
                                 + plsc.load_gather(off_v, [dstg]))
        return 0

    lax.fori_loop(0, GROUPS_A, add_off, 0)
    pltpu.sync_copy(dst_v, dst_sh.at[pl.ds(tbase, TOK_A)])
    pltpu.sync_copy(pos_v, pos_sh.at[pl.ds(tbase, TOK_A)])

    maxc = jnp.max(tot)
    cap = jnp.where(maxc <= 128, 128,
          jnp.where(maxc <= 256, 256,
          jnp.where(maxc <= 512, 512, CAPMAX))).astype(jnp.int32)
    cvec = jnp.minimum(tot, cap)

    plsc.subcore_barrier()

    wid = cid * 16 + sid
    base = wid * ROWS_B
    e = wid // (CAPMAX // ROWS_B)
    h0 = (wid % (CAPMAX // ROWS_B)) * ROWS_B
    ev = jnp.full((L,), e, jnp.int32)
    h0v = jnp.full((L,), h0, jnp.int32)
    v = jnp.sum(jnp.where(iota == ev, jnp.clip(cvec - h0v, 0, ROWS_B), zvec))
    nch = (v + K - 1) // K

    nzb = (ROWS_B - nch * K) // K

    def zdesc(j):
        start = pl.multiple_of(base + (nch + j) * K, 8)
        return pltpu.make_async_copy(zbuf, out_hbm.at[pl.ds(start, K)], zsem)

    def zfire(j, _):
        zdesc(j).start()
        return 0

    lax.fori_loop(0, nzb, zfire, 0)

    pltpu.sync_copy(dst_sh, dst_all)
    pltpu.sync_copy(pos_sh, pos_all)

    def initp(i, _):
        perm_v[pl.ds(i * L, L)] = zvec
        return 0

    lax.fori_loop(0, ROWS_B // L, initp, 0)

    def scan(g, _):
        t0 = g * L
        dstg = dst_all[pl.ds(t0, L)]
        posg = pos_all[pl.ds(t0, L)]
        rel = posg - h0v
        mask = (dstg == ev) & (rel >= 0) & (rel < ROWS_B)
        plsc.store_scatter(perm_v, [rel], iota + t0, mask=mask)
        return 0

    lax.fori_loop(0, N // L, scan, 0)

    KH = K // 2

    def gdescs(c, buf, semg):
        return [
            pltpu.make_async_copy(
                x_hbm.at[perm_v.at[pl.ds(c * K + h * KH, KH)]],
                buf.at[pl.ds(h * KH, KH)], semg)
            for h in range(2)
        ]

    def gstart(c, buf, semg):
        for d in gdescs(c, buf, semg):
            d.start()

    def gwait(c, buf, semg):
        for d in gdescs(c, buf, semg):
            d.wait()

    def sdesc(c, buf, sems):
        start = pl.multiple_of(base + c * K, 8)
        return pltpu.make_async_copy(buf, out_hbm.at[pl.ds(start, K)], sems)

    @pl.when(nch > 0)
    def _():
        gstart(0, buf0, semg0)

    def gstep(c, buf, semg, sems, obuf, osemg, osems):
        @pl.when(c >= 1)
        def _():
            sdesc(c - 1, obuf, osems).wait()

        @pl.when(c + 1 < nch)
        def _():
            gstart(c + 1, obuf, osemg)

        gwait(c, buf, semg)
        sdesc(c, buf, sems).start()

    def gbody(c, _):
        @pl.when(c % 2 == 0)
        def _():
            gstep(c, buf0, semg0, sems0, buf1, semg1, sems1)

        @pl.when(c % 2 == 1)
        def _():
            gstep(c, buf1, semg1, sems1, buf0, semg0, sems0)

        return 0

    lax.fori_loop(0, nch, gbody, 0)

    @pl.when((nch >= 1) & (nch % 2 == 1))
    def _():
        sdesc(nch - 1, buf0, sems0).wait()

    @pl.when((nch >= 1) & (nch % 2 == 0))
    def _():
        sdesc(nch - 1, buf1, sems1).wait()

    @pl.when(v < nch * K)
    def _():
        zidx_v[...] = jnp.minimum(base + v + iota, base + nch * K - 1)
        pltpu.async_copy(zbuf, out_hbm.at[zidx_v], zsem).wait()

    def zdrain(j, _):
        zdesc(j).wait()
        return 0

    lax.fori_loop(0, nzb, zdrain, 0)


_fused = functools.partial(
    pl.kernel,
    out_type=jax.ShapeDtypeStruct((R, D), jnp.float32),
    mesh=_mesh,
    scratch_types=[
        pltpu.VMEM((TOK_A * E,), jnp.float32),
        pltpu.VMEM((TOK_A,), jnp.int32),
        pltpu.VMEM((TOK_A,), jnp.int32),
        pltpu.VMEM((L,), jnp.int32),
        pltpu.VMEM((L,), jnp.int32),
        pltpu.VMEM((L * L,), jnp.int32),
        pltpu.VMEM((N,), jnp.int32),
        pltpu.VMEM((N,), jnp.int32),
        pltpu.VMEM((ROWS_B,), jnp.int32),
        pltpu.VMEM((K, D), jnp.float32),
        pltpu.VMEM((K, D), jnp.float32),
        pltpu.VMEM((L, D), jnp.float32),
        pltpu.VMEM((L,), jnp.int32),
        pltpu.VMEM_SHARED((L * L,), jnp.int32),
        pltpu.VMEM_SHARED((N,), jnp.int32),
        pltpu.VMEM_SHARED((N,), jnp.int32),
        pltpu.SemaphoreType.DMA,
        pltpu.SemaphoreType.DMA,
        pltpu.SemaphoreType.DMA,
        pltpu.SemaphoreType.DMA,
        pltpu.SemaphoreType.DMA,
    ],
    compiler_params=_params,
)(_fused_body)


def kernel(x, route_logits):
    zsrc = jnp.zeros((L, D), jnp.float32)
    out = _fused(route_logits.reshape(N * E), x, zsrc)
    return out.reshape(E, CAPMAX, D)


# speedup vs baseline: 1.1268x; 1.0101x over previous
import functools

import jax
import jax.numpy as jnp
from jax import lax
from jax.experimental import pallas as pl
from jax.experimental.pallas import tpu as pltpu
from jax.experimental.pallas import tpu_sc as plsc

N = 8192
D = 2048
E = 16
CAPMAX = 1024
R = E * CAPMAX
L = 16
NTILES = 32
TOK_A = N // L
GROUPS_A = TOK_A // L
ROWS_B = R // NTILES
K = 16

_mesh = plsc.VectorSubcoreMesh(core_axis_name="c", subcore_axis_name="s",
                               num_cores=2, num_subcores=16)
_params = pltpu.CompilerParams(needs_layout_passes=False)


def _fused_body(logits_hbm, x_hbm, zsrc_hbm, out_hbm,
                logits_v, dst_v, pos_v, cnt_v, off_v, counts_v,
                dst_all, pos_all, perm_v, buf0, buf1, zbuf, zidx_v,
                counts_sh, dst_sh, pos_sh,
                semg0, semg1, sems0, sems1, zsem):
    cid = lax.axis_index("c")
    sid = lax.axis_index("s")
    iota = lax.iota(jnp.int32, L)
    zvec = jnp.zeros((L,), jnp.int32)

    tbase = sid * TOK_A
    pltpu.sync_copy(logits_hbm.at[pl.ds(tbase * E, TOK_A * E)], logits_v)
    pltpu.sync_copy(zsrc_hbm, zbuf)
    cnt_v[...] = zvec

    def group(g, _):
        row0 = g * L
        flat = (iota + row0) * E
        cols = [plsc.load_gather(logits_v, [flat + d]) for d in range(E)]
        m = cols[0]
        for d in range(1, E):
            m = jnp.maximum(m, cols[d])
        dst = jnp.full((L,), E, jnp.int32)
        for d in range(E - 1, -1, -1):
            dst = jnp.where(cols[d] == m, d, dst)
        basec = plsc.load_gather(cnt_v, [dst])
        rank = zvec
        inc = zvec
        for d in range(E):
            md = dst == d
            mi = md.astype(jnp.int32)
            c = plsc.cumsum(mi)
            rank = jnp.where(md, rank + c - 1, rank)
            inc = jnp.where(iota == d, jnp.sum(mi), inc)
        cnt_v[...] = cnt_v[...] + inc
        dst_v[pl.ds(row0, L)] = dst
        pos_v[pl.ds(row0, L)] = basec + rank
        return 0

    lax.fori_loop(0, GROUPS_A, group, 0)

    pltpu.sync_copy(cnt_v, counts_sh.at[pl.ds(sid * L, L)])
    plsc.subcore_barrier()
    pltpu.sync_copy(counts_sh, counts_v)

    off = zvec
    tot = zvec
    sidv = jnp.full((L,), sid, jnp.int32)
    for t in range(L):
        row = counts_v[pl.ds(t * L, L)]
        off = off + jnp.where(sidv > t, row, zvec)
        tot = tot + row
    off_v[...] = off

    def add_off(g, _):
        row0 = g * L
        dstg = dst_v[pl.ds(row0, L)]
        pos_v[pl.ds(row0, L)] = (pos_v[pl.ds(row0, L)]
                                 + plsc.load_gather(off_v, [dstg]))
        return 0

    lax.fori_loop(0, GROUPS_A, add_off, 0)
    pltpu.sync_copy(dst_v, dst_sh.at[pl.ds(tbase, TOK_A)])
    pltpu.sync_copy(pos_v, pos_sh.at[pl.ds(tbase, TOK_A)])

    maxc = jnp.max(tot)
    cap = jnp.where(maxc <= 128, 128,
          jnp.where(maxc <= 256, 256,
          jnp.where(maxc <= 512, 512, CAPMAX))).astype(jnp.int32)
    cvec = jnp.minimum(tot, cap)

    plsc.subcore_barrier()

    wid = cid * 16 + sid
    base = wid * ROWS_B
    e = wid // (CAPMAX // ROWS_B)
    h0 = (wid % (CAPMAX // ROWS_B)) * ROWS_B
    ev = jnp.full((L,), e, jnp.int32)
    h0v = jnp.full((L,), h0, jnp.int32)
    v = jnp.sum(jnp.where(iota == ev, jnp.clip(cvec - h0v, 0, ROWS_B), zvec))
    nch = (v + K - 1) // K

    nzb = (ROWS_B - nch * K) // K

    def zdesc(j):
        start = pl.multiple_of(base + (nch + j) * K, 8)
        return pltpu.make_async_copy(zbuf, out_hbm.at[pl.ds(start, K)], zsem)

    def zfire(j, _):
        zdesc(j).start()
        return 0

    lax.fori_loop(0, nzb, zfire, 0)

    pltpu.sync_copy(dst_sh, dst_all)
    pltpu.sync_copy(pos_sh, pos_all)

    def initp(i, _):
        perm_v[pl.ds(i * L, L)] = zvec
        return 0

    lax.fori_loop(0, ROWS_B // L, initp, 0)

    def scan(g, _):
        t0 = g * L
        dstg = dst_all[pl.ds(t0, L)]
        posg = pos_all[pl.ds(t0, L)]
        rel = posg - h0v
        mask = (dstg == ev) & (rel >= 0) & (rel < ROWS_B)
        plsc.store_scatter(perm_v, [rel], iota + t0, mask=mask)
        return 0

    lax.fori_loop(0, N // L, scan, 0)

    def gdesc(c, buf, semg):
        pvals = perm_v[pl.ds(c * K, K)]
        return pltpu.make_async_copy(x_hbm.at[pvals], buf, semg)

    def gstart(c, buf, semg):
        gdesc(c, buf, semg).start()

    def gwait(c, buf, semg):
        gdesc(c, buf, semg).wait()

    def sdesc(c, buf, sems):
        start = pl.multiple_of(base + c * K, 8)
        return pltpu.make_async_copy(buf, out_hbm.at[pl.ds(start, K)], sems)

    @pl.when(nch > 0)
    def _():
        gstart(0, buf0, semg0)

    def gstep(c, buf, semg, sems, obuf, osemg, osems):
        @pl.when(c >= 1)
        def _():
            sdesc(c - 1, obuf, osems).wait()

        @pl.when(c + 1 < nch)
        def _():
            gstart(c + 1, obuf, osemg)

        gwait(c, buf, semg)
        sdesc(c, buf, sems).start()

    def gbody(c, _):
        @pl.when(c % 2 == 0)
        def _():
            gstep(c, buf0, semg0, sems0, buf1, semg1, sems1)

        @pl.when(c % 2 == 1)
        def _():
            gstep(c, buf1, semg1, sems1, buf0, semg0, sems0)

        return 0

    lax.fori_loop(0, nch, gbody, 0)

    @pl.when((nch >= 1) & (nch % 2 == 1))
    def _():
        sdesc(nch - 1, buf0, sems0).wait()

    @pl.when((nch >= 1) & (nch % 2 == 0))
    def _():
        sdesc(nch - 1, buf1, sems1).wait()

    @pl.when(v < nch * K)
    def _():
        zidx_v[...] = jnp.minimum(base + v + iota, base + nch * K - 1)
        pltpu.async_copy(zbuf, out_hbm.at[zidx_v], zsem).wait()

    def zdrain(j, _):
        zdesc(j).wait()
        return 0

    lax.fori_loop(0, nzb, zdrain, 0)


_fused = functools.partial(
    pl.kernel,
    out_type=jax.ShapeDtypeStruct((R, D), jnp.float32),
    mesh=_mesh,
    scratch_types=[
        pltpu.VMEM((TOK_A * E,), jnp.float32),
        pltpu.VMEM((TOK_A,), jnp.int32),
        pltpu.VMEM((TOK_A,), jnp.int32),
        pltpu.VMEM((L,), jnp.int32),
        pltpu.VMEM((L,), jnp.int32),
        pltpu.VMEM((L * L,), jnp.int32),
        pltpu.VMEM((N,), jnp.int32),
        pltpu.VMEM((N,), jnp.int32),
        pltpu.VMEM((ROWS_B,), jnp.int32),
        pltpu.VMEM((K, D), jnp.float32),
        pltpu.VMEM((K, D), jnp.float32),
        pltpu.VMEM((L, D), jnp.float32),
        pltpu.VMEM((L,), jnp.int32),
        pltpu.VMEM_SHARED((L * L,), jnp.int32),
        pltpu.VMEM_SHARED((N,), jnp.int32),
        pltpu.VMEM_SHARED((N,), jnp.int32),
        pltpu.SemaphoreType.DMA,
        pltpu.SemaphoreType.DMA,
        pltpu.SemaphoreType.DMA,
        pltpu.SemaphoreType.DMA,
        pltpu.SemaphoreType.DMA,
    ],
    compiler_params=_params,
)(_fused_body)


def kernel(x, route_logits):
    zsrc = jnp.zeros((L, D), jnp.float32)
    out = _fused(route_logits.reshape(N * E), x, zsrc)
    return out.reshape(E, CAPMAX, D)
